# baseline (speedup 1.0000x reference)
import jax
import jax.numpy as jnp
from jax import lax
from jax.experimental import pallas as pl
from jax.experimental.pallas import tpu as pltpu
from jax.experimental.pallas import tpu_sc as plsc

VOCAB = 1000000
DIM = 64
BATCH = 4096
SEQ = 50

NC = 2
NS = 16
L = 16
NW = NC * NS
ROWS_PER_W = BATCH // NW
CB = 16
TOK_CB = CB * SEQ
GROUPS = 8
G_TOK = TOK_CB // GROUPS
N_CHUNKS = ROWS_PER_W // CB


def _rsqrt(x):
    i = lax.bitcast_convert_type(x, jnp.int32)
    y = lax.bitcast_convert_type(jnp.int32(0x5F3759DF) - (i >> 1), jnp.float32)
    for _ in range(3):
        y = y * (1.5 - 0.5 * x * y * y)
    return y


def _body(tok_hbm, msk_hbm, embed_hbm, lnw_hbm, lnb_hbm, out_hbm,
          idx_v, msk_v, mskf_v, rows_v, out_v, lnw_v, lnb_v, sem):
    wid = lax.axis_index("s") * NC + lax.axis_index("c")
    pltpu.sync_copy(lnw_hbm, lnw_v)
    pltpu.sync_copy(lnb_hbm, lnb_v)

    @pl.loop(0, N_CHUNKS)
    def _chunk(c):
        row0 = pl.multiple_of(wid * ROWS_PER_W + c * CB, 16)
        tok0 = pl.multiple_of(row0 // 2, 8)
        flat0 = pl.multiple_of(row0 * SEQ, 8)

        pltpu.sync_copy(tok_hbm.at[pl.ds(tok0, GROUPS)], idx_v)
        pltpu.sync_copy(msk_hbm.at[pl.ds(flat0, TOK_CB)], msk_v)
        copies = [
            pltpu.async_copy(embed_hbm.at[idx_v.at[g]],
                             rows_v.at[pl.ds(g * G_TOK, G_TOK)], sem)
            for g in range(GROUPS)
        ]
        for t in range(TOK_CB // L):
            mskf_v[pl.ds(t * L, L)] = msk_v[pl.ds(t * L, L)].astype(jnp.float32)
        for cp in copies:
            cp.wait()

        for b in range(CB):
            base = b * SEQ

            def _tok(s, carry):
                a0, a1, a2, a3, cnt = carry
                w = plsc.load_gather(
                    mskf_v, [jnp.full((L,), base + s, jnp.int32)])
                a0 = a0 + w * rows_v[base + s, pl.ds(0, L)]
                a1 = a1 + w * rows_v[base + s, pl.ds(L, L)]
                a2 = a2 + w * rows_v[base + s, pl.ds(2 * L, L)]
                a3 = a3 + w * rows_v[base + s, pl.ds(3 * L, L)]
                return a0, a1, a2, a3, cnt + w

            z = jnp.zeros((L,), jnp.float32)
            a0, a1, a2, a3, cnt = lax.fori_loop(
                0, SEQ, _tok, (z, z, z, z, z), unroll=5)

            inv = 1.0 / jnp.maximum(cnt, 1.0)
            p0, p1, p2, p3 = a0 * inv, a1 * inv, a2 * inv, a3 * inv
            m = jnp.sum(p0 + p1 + p2 + p3) * (1.0 / DIM)
            d0, d1, d2, d3 = p0 - m, p1 - m, p2 - m, p3 - m
            var = jnp.sum(d0 * d0 + d1 * d1 + d2 * d2 + d3 * d3) * (1.0 / DIM)
            r = _rsqrt(jnp.full((L,), 1.0, jnp.float32) * (var + 1e-5))
            out_v[b, pl.ds(0, L)] = d0 * r * lnw_v[pl.ds(0, L)] + lnb_v[pl.ds(0, L)]
            out_v[b, pl.ds(L, L)] = d1 * r * lnw_v[pl.ds(L, L)] + lnb_v[pl.ds(L, L)]
            out_v[b, pl.ds(2 * L, L)] = d2 * r * lnw_v[pl.ds(2 * L, L)] + lnb_v[pl.ds(2 * L, L)]
            out_v[b, pl.ds(3 * L, L)] = d3 * r * lnw_v[pl.ds(3 * L, L)] + lnb_v[pl.ds(3 * L, L)]

        pltpu.sync_copy(out_v, out_hbm.at[pl.ds(row0, CB)])


@jax.jit
def _encoder(tok2d, mskflat, embed, ln_weight, ln_bias):
    mesh = plsc.VectorSubcoreMesh(core_axis_name="c", subcore_axis_name="s",
                                  num_cores=NC, num_subcores=NS)
    return pl.kernel(
        _body,
        out_type=jax.ShapeDtypeStruct((BATCH, DIM), jnp.float32),
        mesh=mesh,
        compiler_params=pltpu.CompilerParams(needs_layout_passes=False,
                                             use_tc_tiling_on_sc=False),
        scratch_types=[
            pltpu.VMEM((GROUPS, G_TOK), jnp.int32),
            pltpu.VMEM((TOK_CB,), jnp.int32),
            pltpu.VMEM((TOK_CB,), jnp.float32),
            pltpu.VMEM((TOK_CB, DIM), jnp.float32),
            pltpu.VMEM((CB, DIM), jnp.float32),
            pltpu.VMEM((DIM,), jnp.float32),
            pltpu.VMEM((DIM,), jnp.float32),
            pltpu.SemaphoreType.DMA,
        ],
    )(tok2d, mskflat, embed, ln_weight, ln_bias)


def kernel(token_ids, attention_mask, embed, ln_weight, ln_bias):
    tok2d = token_ids.astype(jnp.int32).reshape(-1, G_TOK)
    mskflat = attention_mask.astype(jnp.int32).reshape(-1)
    return _encoder(tok2d, mskflat, embed, ln_weight, ln_bias)


# speedup vs baseline: 1.0045x; 1.0045x over previous
import jax
import jax.numpy as jnp
from jax import lax
from jax.experimental import pallas as pl
from jax.experimental.pallas import tpu as pltpu
from jax.experimental.pallas import tpu_sc as plsc

VOCAB = 1000000
DIM = 64
BATCH = 4096
SEQ = 50

NC = 2
NS = 16
L = 16
NW = NC * NS
ROWS_PER_W = BATCH // NW
CB = 16
TOK_CB = CB * SEQ
N_CHUNKS = ROWS_PER_W // CB


def _rsqrt(x):
    i = lax.bitcast_convert_type(x, jnp.int32)
    y = lax.bitcast_convert_type(jnp.int32(0x5F3759DF) - (i >> 1), jnp.float32)
    for _ in range(3):
        y = y * (1.5 - 0.5 * x * y * y)
    return y


def _body(tok_hbm, msk_hbm, embed_hbm, lnw_hbm, lnb_hbm, out_hbm,
          idx_v, msk_v, rows_v, out_v, lnw_v, lnb_v, sem):
    wid = lax.axis_index("s") * NC + lax.axis_index("c")
    pltpu.sync_copy(lnw_hbm, lnw_v)
    pltpu.sync_copy(lnb_hbm, lnb_v)

    @pl.loop(0, N_CHUNKS)
    def _chunk(c):
        row0 = pl.multiple_of(wid * ROWS_PER_W + c * CB, 16)

        pltpu.sync_copy(tok_hbm.at[pl.ds(row0, CB)], idx_v)
        pltpu.sync_copy(msk_hbm.at[pl.ds(row0, CB)], msk_v)
        copies = [
            pltpu.async_copy(embed_hbm.at[idx_v.at[b]],
                             rows_v.at[pl.ds(b * SEQ, SEQ)], sem)
            for b in range(CB)
        ]
        for cp in copies:
            cp.wait()

        for b in range(CB):
            base = b * SEQ
            bvec = jnp.full((L,), b, jnp.int32)

            def _tok(s, carry):
                a0, a1, a2, a3, cnt = carry
                w = plsc.load_gather(
                    msk_v, [bvec, jnp.full((L,), s, jnp.int32)]
                ).astype(jnp.float32)
                a0 = a0 + w * rows_v[base + s, pl.ds(0, L)]
                a1 = a1 + w * rows_v[base + s, pl.ds(L, L)]
                a2 = a2 + w * rows_v[base + s, pl.ds(2 * L, L)]
                a3 = a3 + w * rows_v[base + s, pl.ds(3 * L, L)]
                return a0, a1, a2, a3, cnt + w

            z = jnp.zeros((L,), jnp.float32)
            a0, a1, a2, a3, cnt = lax.fori_loop(
                0, SEQ, _tok, (z, z, z, z, z), unroll=5)

            inv = 1.0 / jnp.maximum(cnt, 1.0)
            p0, p1, p2, p3 = a0 * inv, a1 * inv, a2 * inv, a3 * inv
            m = jnp.sum(p0 + p1 + p2 + p3) * (1.0 / DIM)
            d0, d1, d2, d3 = p0 - m, p1 - m, p2 - m, p3 - m
            var = jnp.sum(d0 * d0 + d1 * d1 + d2 * d2 + d3 * d3) * (1.0 / DIM)
            r = _rsqrt(jnp.full((L,), 1.0, jnp.float32) * (var + 1e-5))
            out_v[b, pl.ds(0, L)] = d0 * r * lnw_v[pl.ds(0, L)] + lnb_v[pl.ds(0, L)]
            out_v[b, pl.ds(L, L)] = d1 * r * lnw_v[pl.ds(L, L)] + lnb_v[pl.ds(L, L)]
            out_v[b, pl.ds(2 * L, L)] = d2 * r * lnw_v[pl.ds(2 * L, L)] + lnb_v[pl.ds(2 * L, L)]
            out_v[b, pl.ds(3 * L, L)] = d3 * r * lnw_v[pl.ds(3 * L, L)] + lnb_v[pl.ds(3 * L, L)]

        pltpu.sync_copy(out_v, out_hbm.at[pl.ds(row0, CB)])


@jax.jit
def _encoder(tok, msk, embed, ln_weight, ln_bias):
    mesh = plsc.VectorSubcoreMesh(core_axis_name="c", subcore_axis_name="s",
                                  num_cores=NC, num_subcores=NS)
    return pl.kernel(
        _body,
        out_type=jax.ShapeDtypeStruct((BATCH, DIM), jnp.float32),
        mesh=mesh,
        compiler_params=pltpu.CompilerParams(needs_layout_passes=False,
                                             use_tc_tiling_on_sc=False),
        scratch_types=[
            pltpu.VMEM((CB, SEQ), jnp.int32),
            pltpu.VMEM((CB, SEQ), jnp.int32),
            pltpu.VMEM((TOK_CB, DIM), jnp.float32),
            pltpu.VMEM((CB, DIM), jnp.float32),
            pltpu.VMEM((DIM,), jnp.float32),
            pltpu.VMEM((DIM,), jnp.float32),
            pltpu.SemaphoreType.DMA,
        ],
    )(tok, msk, embed, ln_weight, ln_bias)


def kernel(token_ids, attention_mask, embed, ln_weight, ln_bias):
    return _encoder(token_ids.astype(jnp.int32),
                    attention_mask.astype(jnp.int32),
                    embed, ln_weight, ln_bias)
